# Initial kernel scaffold; baseline (speedup 1.0000x reference)
import jax
import jax.numpy as jnp
from jax.experimental import pallas as pl


def kernel(x_nodes, x_commodities, x_edges_capacity, x_edges_usage, W_node, b_node, W_comm, b_comm, W_edge, b_edge, W_msg, b_msg, W_nupd, b_nupd, W_eupd, b_eupd, W_graph, b_graph):
    raise NotImplementedError("write your pallas kernel here")


# speedup vs baseline: 4.2240x; 4.2240x over previous
import jax
import jax.numpy as jnp
from jax.experimental import pallas as pl

V = 24
C = 32
H = 256
L = 3


def _body(xn_ref, xc_ref, cap_ref, usage_ref,
          Wn_ref, bn_ref, Wc_ref, bc_ref, We_ref, be_ref,
          Wm_ref, bm_ref, Wnu_ref, bnu_ref, Weu_ref, beu_ref,
          Wg_ref, bg_ref,
          h_out, he_out, g_out):
    f32 = jnp.float32
    bf16 = jnp.bfloat16
    relu = jax.nn.relu

    def mm(x, w):
        return jnp.dot(x.astype(bf16), w.astype(bf16),
                       preferred_element_type=f32)

    comm = mm(xc_ref[0], Wc_ref[...]) + bc_ref[...][None, :]
    xn = xn_ref[0].reshape(V * C, 4)
    h = relu(mm(xn, Wn_ref[...]) + bn_ref[...][None, :]
             + jnp.tile(comm, (V, 1)))

    we0 = We_ref[0][None, None, None, :]
    we1 = We_ref[1][None, None, None, :]
    be = be_ref[...][None, None, None, :]

    S = 6
    agg = jnp.zeros((V, C, H), f32)
    for s0 in range(0, V, S):
        cap_s = cap_ref[0, s0:s0 + S]
        use_s = usage_ref[0, s0:s0 + S]
        e = relu(cap_s[:, :, None, None] * we0
                 + use_s[:, :, :, None] * we1 + be)
        he_out[0, s0:s0 + S] = e
        agg = agg + e.sum(axis=0)

    for l in range(L):
        aggm = (agg * (1.0 / V)).reshape(V * C, H)
        msg = relu(mm(aggm, Wm_ref[l]) + bm_ref[l][None, :])
        h = relu(mm(h, Wnu_ref[l, :H]) + mm(msg, Wnu_ref[l, H:])
                 + bnu_ref[l][None, :]) + h
        a = mm(h, Weu_ref[l, :H]).reshape(V, C, H)
        d = mm(h, Weu_ref[l, H:2 * H]).reshape(V, C, H)
        we3 = Weu_ref[l, 2 * H:].astype(bf16)
        beu = beu_ref[l][None, None, None, :]

        agg = jnp.zeros((V, C, H), f32)
        for s0 in range(0, V, S):
            he_s = he_out[0, s0:s0 + S]
            p = jnp.dot(he_s.reshape(S * V * C, H).astype(bf16), we3,
                        preferred_element_type=f32).reshape(S, V, C, H)
            up = relu(p + a[s0:s0 + S, None, :, :] + d[None] + beu) + he_s
            he_out[0, s0:s0 + S] = up
            agg = agg + up.sum(axis=0)

    h_out[0] = h.reshape(V, C, H)
    gm = jnp.mean(h, axis=0, keepdims=True)
    g_out[0] = mm(gm, Wg_ref[...]) + bg_ref[...][None, :]


def kernel(x_nodes, x_commodities, x_edges_capacity, x_edges_usage,
           W_node, b_node, W_comm, b_comm, W_edge, b_edge,
           W_msg, b_msg, W_nupd, b_nupd, W_eupd, b_eupd,
           W_graph, b_graph):
    B = x_nodes.shape[0]
    f32 = jnp.float32

    full = lambda shape: pl.BlockSpec(shape, lambda b: (0,) * len(shape))
    out_shapes = (
        jax.ShapeDtypeStruct((B, V, C, H), f32),
        jax.ShapeDtypeStruct((B, V, V, C, H), f32),
        jax.ShapeDtypeStruct((B, 1, H), f32),
    )
    h, he, g = pl.pallas_call(
        _body,
        grid=(B,),
        in_specs=[
            pl.BlockSpec((1, V, C, 4), lambda b: (b, 0, 0, 0)),
            pl.BlockSpec((1, C, 3), lambda b: (b, 0, 0)),
            pl.BlockSpec((1, V, V), lambda b: (b, 0, 0)),
            pl.BlockSpec((1, V, V, C), lambda b: (b, 0, 0, 0)),
            full((4, H)), full((H,)),
            full((3, H)), full((H,)),
            full((2, H)), full((H,)),
            full((L, H, H)), full((L, H)),
            full((L, 2 * H, H)), full((L, H)),
            full((L, 3 * H, H)), full((L, H)),
            full((H, H)), full((H,)),
        ],
        out_specs=[
            pl.BlockSpec((1, V, C, H), lambda b: (b, 0, 0, 0)),
            pl.BlockSpec((1, V, V, C, H), lambda b: (b, 0, 0, 0, 0)),
            pl.BlockSpec((1, 1, H), lambda b: (b, 0, 0)),
        ],
        out_shape=out_shapes,
    )(x_nodes, x_commodities, x_edges_capacity, x_edges_usage,
      W_node, b_node, W_comm, b_comm, W_edge, b_edge,
      W_msg, b_msg, W_nupd, b_nupd, W_eupd, b_eupd,
      W_graph, b_graph)
    return (h, he, g.reshape(B, H))
